# baseline (speedup 1.0000x reference)
import functools

import jax
import jax.numpy as jnp
from jax import lax
from jax.experimental import pallas as pl
from jax.experimental.pallas import tpu as pltpu
from jax.experimental.pallas import tpu_sc as plsc

NUM_EMB = 1000000
D = 32
BATCH = 4096
HIST = 200
TOTAL = BATCH * HIST

NC = 2
NS = 16
NW = NC * NS
PER_W = TOTAL // NW

G = 128
CHUNK = 1280
N_G = CHUNK // G
N_IT = PER_W // CHUNK
IDX_ROWS_W = PER_W // G

_CIN = 16384
_CIN_GRID = -(-NUM_EMB // _CIN)


_Q = _CIN // 4
_NEMB_PAD = _CIN_GRID * _CIN


def _conv_in_body(in_ref, o_ref):
    eye = jnp.eye(D, dtype=jnp.float32)
    t = lax.dot_general(in_ref[...], eye, (((0,), (0,)), ((), ())),
                        precision=lax.Precision.HIGHEST)
    for k0 in range(4):
        o_ref[:, k0 * 32:(k0 + 1) * 32] = t[k0 * _Q:(k0 + 1) * _Q, :]


_conv_in = pl.pallas_call(
    _conv_in_body,
    grid=(_CIN_GRID,),
    in_specs=[pl.BlockSpec((D, _CIN), lambda i: (0, i))],
    out_specs=pl.BlockSpec((_Q, 128), lambda i: (i, 0)),
    out_shape=jax.ShapeDtypeStruct((_NEMB_PAD * D // 128, 128), jnp.float32),
)

_BROWS = TOTAL * D // 128 // NW


def _conv_out_body(in_ref, o_ref):
    def one(h4, carry):
        blk = in_ref[pl.Slice(h4, 128, HIST * D // 128), :]
        blkT = blk.T
        for j in range(4):
            o_ref[h4 * 4 + j, :, 0] = (
                blkT[j * 32:(j + 1) * 32, :].reshape(4, 8, 128))
        return carry

    lax.fori_loop(0, HIST // 4, one, 0)


_conv_out_a = pl.pallas_call(
    _conv_out_body,
    grid=(NW // 2,),
    in_specs=[pl.BlockSpec((_BROWS, 128), lambda i: (i, 0))],
    out_specs=pl.BlockSpec((HIST, 4, 1, 8, 128), lambda i: (0, 0, i, 0, 0)),
    out_shape=jax.ShapeDtypeStruct((HIST, 4, NW, 8, 128), jnp.float32),
)


def _conv_out_body_b(in_ref, prev_ref, o_ref):
    del prev_ref
    _conv_out_body(in_ref, o_ref)


_conv_out_b = pl.pallas_call(
    _conv_out_body_b,
    grid=(NW // 2,),
    in_specs=[pl.BlockSpec((_BROWS, 128), lambda i: (i, 0)),
              pl.BlockSpec(memory_space=pl.ANY)],
    out_specs=pl.BlockSpec((HIST, 4, 1, 8, 128),
                           lambda i: (0, 0, NW // 2 + i, 0, 0)),
    out_shape=jax.ShapeDtypeStruct((HIST, 4, NW, 8, 128), jnp.float32),
    input_output_aliases={1: 0},
)


_HTOT = TOTAL // 2
_HPER_W = _HTOT // NW
_HN_IT = _HPER_W // CHUNK
_HIDX_ROWS = _HPER_W // G


def _build(half):
    mesh = plsc.VectorSubcoreMesh(core_axis_name="c", subcore_axis_name="s")

    @functools.partial(
        pl.kernel,
        mesh=mesh,
        out_type=jax.ShapeDtypeStruct((_HTOT, D), jnp.float32),
        scratch_types=[
            pltpu.VMEM((_HIDX_ROWS, G), jnp.int32),
            pltpu.VMEM((CHUNK, D), jnp.float32),
            pltpu.VMEM((CHUNK, D), jnp.float32),
            pltpu.SemaphoreType.DMA,
            pltpu.SemaphoreType.DMA,
            pltpu.SemaphoreType.DMA,
            pltpu.SemaphoreType.DMA,
        ],
        compiler_params=pltpu.CompilerParams(use_tc_tiling_on_sc=False),
    )
    def body(idx_hbm, table_hbm, out_hbm, idx_all, rows0, rows1,
             sg0, sg1, so0, so1):
        rows = (rows0, rows1)
        sem_g = (sg0, sg1)
        sem_o = (so0, so1)
        wid = lax.axis_index("s") * NC + lax.axis_index("c")
        row_base = wid * _HPER_W
        idx_row0 = half * (_HTOT // G) + wid * _HIDX_ROWS

        pltpu.sync_copy(idx_hbm.at[pl.ds(idx_row0, _HIDX_ROWS)], idx_all)

        def permute_idx(r, carry):
            for j in range(G // 16):
                v = idx_all[r, pl.ds(j * 16, 16)]
                idx_all[r, pl.ds(j * 16, 16)] = (
                    (v & -16384) | ((v & 4095) << 2) | ((v >> 12) & 3))
            return carry

        lax.fori_loop(0, _HIDX_ROWS, permute_idx, 0)

        def fire(g, slot):
            for j in range(N_G):
                pltpu.async_copy(table_hbm.at[idx_all.at[g * N_G + j]],
                                 rows[slot].at[pl.ds(j * G, G)],
                                 sem_g[slot])

        def drain_gathers(slot):
            for j in range(N_G):
                pltpu.make_async_copy(table_hbm.at[idx_all.at[0]],
                                      rows[slot].at[pl.ds(j * G, G)],
                                      sem_g[slot]).wait()

        def wait_store(slot):
            pltpu.make_async_copy(rows[slot],
                                  out_hbm.at[pl.ds(row_base, CHUNK)],
                                  sem_o[slot]).wait()

        fire(0, 0)

        def it(i, carry):
            for k in range(2):
                g = 2 * i + k
                slot = k
                nxt = 1 - k

                @pl.when(g >= 1)
                def _():
                    wait_store(nxt)

                @pl.when(g + 1 < _HN_IT)
                def _():
                    fire(g + 1, nxt)

                drain_gathers(slot)
                pltpu.async_copy(rows[slot],
                                 out_hbm.at[pl.ds(row_base + g * CHUNK,
                                                  CHUNK)],
                                 sem_o[slot])
            return carry

        lax.fori_loop(0, _HN_IT // 2, it, 0)
        wait_store((_HN_IT - 1) % 2)

    return body


_gather_a = _build(0)
_gather_b = _build(1)


def kernel(input_seq, weights):
    idx = input_seq.reshape(TOTAL // G, G)
    w128 = _conv_in(weights.T)
    w_lin = w128.reshape(_NEMB_PAD, D)
    ga = _gather_a(idx, w_lin)
    gb = _gather_b(idx, w_lin)
    oa = _conv_out_a(ga.reshape(_HTOT * D // 128, 128))
    out5 = _conv_out_b(gb.reshape(_HTOT * D // 128, 128), oa)
    return out5.transpose(2, 4, 0, 1, 3).reshape(BATCH, HIST, D)


# speedup vs baseline: 1.3860x; 1.3860x over previous
import functools

import jax
import jax.numpy as jnp
from jax import lax
from jax.experimental import pallas as pl
from jax.experimental.pallas import tpu as pltpu
from jax.experimental.pallas import tpu_sc as plsc

NUM_EMB = 1000000
D = 32
BATCH = 4096
HIST = 200
TOTAL = BATCH * HIST

NC = 2
NS = 16
NW = NC * NS
PER_W = TOTAL // NW

G = 128
CHUNK = 1280
N_G = CHUNK // G
N_IT = PER_W // CHUNK
IDX_ROWS_W = PER_W // G

_CIN = 16384
_CIN_GRID = -(-NUM_EMB // _CIN)


_Q = _CIN // 4
_NEMB_PAD = _CIN_GRID * _CIN


def _conv_in_body(in_ref, o_ref):
    t = in_ref[...].T
    for k0 in range(4):
        o_ref[:, k0 * 32:(k0 + 1) * 32] = t[k0 * _Q:(k0 + 1) * _Q, :]


_conv_in = pl.pallas_call(
    _conv_in_body,
    grid=(_CIN_GRID,),
    in_specs=[pl.BlockSpec((D, _CIN), lambda i: (0, i))],
    out_specs=pl.BlockSpec((_Q, 128), lambda i: (i, 0)),
    out_shape=jax.ShapeDtypeStruct((_NEMB_PAD * D // 128, 128), jnp.float32),
)

_BROWS = TOTAL * D // 128 // NW


def _conv_out_body(in_ref, o_ref):
    def one(h4, carry):
        blk = in_ref[pl.Slice(h4, 128, HIST * D // 128), :]
        blkT = blk.T
        for j in range(4):
            o_ref[h4 * 4 + j, :, 0] = (
                blkT[j * 32:(j + 1) * 32, :].reshape(4, 8, 128))
        return carry

    lax.fori_loop(0, HIST // 4, one, 0)


_conv_out_a = pl.pallas_call(
    _conv_out_body,
    grid=(NW // 2,),
    in_specs=[pl.BlockSpec((_BROWS, 128), lambda i: (i, 0))],
    out_specs=pl.BlockSpec((HIST, 4, 1, 8, 128), lambda i: (0, 0, i, 0, 0)),
    out_shape=jax.ShapeDtypeStruct((HIST, 4, NW, 8, 128), jnp.float32),
)


def _conv_out_body_b(in_ref, prev_ref, o_ref):
    del prev_ref
    _conv_out_body(in_ref, o_ref)


_conv_out_b = pl.pallas_call(
    _conv_out_body_b,
    grid=(NW // 2,),
    in_specs=[pl.BlockSpec((_BROWS, 128), lambda i: (i, 0)),
              pl.BlockSpec(memory_space=pl.ANY)],
    out_specs=pl.BlockSpec((HIST, 4, 1, 8, 128),
                           lambda i: (0, 0, NW // 2 + i, 0, 0)),
    out_shape=jax.ShapeDtypeStruct((HIST, 4, NW, 8, 128), jnp.float32),
    input_output_aliases={1: 0},
)


_HTOT = TOTAL // 2
_HPER_W = _HTOT // NW
_HN_IT = _HPER_W // CHUNK
_HIDX_ROWS = _HPER_W // G


def _build(half):
    mesh = plsc.VectorSubcoreMesh(core_axis_name="c", subcore_axis_name="s")

    @functools.partial(
        pl.kernel,
        mesh=mesh,
        out_type=jax.ShapeDtypeStruct((_HTOT, D), jnp.float32),
        scratch_types=[
            pltpu.VMEM((_HIDX_ROWS, G), jnp.int32),
            pltpu.VMEM((CHUNK, D), jnp.float32),
            pltpu.VMEM((CHUNK, D), jnp.float32),
            pltpu.SemaphoreType.DMA,
            pltpu.SemaphoreType.DMA,
            pltpu.SemaphoreType.DMA,
            pltpu.SemaphoreType.DMA,
        ],
        compiler_params=pltpu.CompilerParams(use_tc_tiling_on_sc=False),
    )
    def body(idx_hbm, table_hbm, out_hbm, idx_all, rows0, rows1,
             sg0, sg1, so0, so1):
        rows = (rows0, rows1)
        sem_g = (sg0, sg1)
        sem_o = (so0, so1)
        wid = lax.axis_index("s") * NC + lax.axis_index("c")
        row_base = wid * _HPER_W
        idx_row0 = half * (_HTOT // G) + wid * _HIDX_ROWS

        pltpu.sync_copy(idx_hbm.at[pl.ds(idx_row0, _HIDX_ROWS)], idx_all)

        def permute_idx(r, carry):
            for j in range(G // 16):
                v = idx_all[r, pl.ds(j * 16, 16)]
                idx_all[r, pl.ds(j * 16, 16)] = (
                    (v & -16384) | ((v & 4095) << 2) | ((v >> 12) & 3))
            return carry

        lax.fori_loop(0, _HIDX_ROWS, permute_idx, 0)

        def fire(g, slot):
            for j in range(N_G):
                pltpu.async_copy(table_hbm.at[idx_all.at[g * N_G + j]],
                                 rows[slot].at[pl.ds(j * G, G)],
                                 sem_g[slot])

        def drain_gathers(slot):
            for j in range(N_G):
                pltpu.make_async_copy(table_hbm.at[idx_all.at[0]],
                                      rows[slot].at[pl.ds(j * G, G)],
                                      sem_g[slot]).wait()

        def wait_store(slot):
            pltpu.make_async_copy(rows[slot],
                                  out_hbm.at[pl.ds(row_base, CHUNK)],
                                  sem_o[slot]).wait()

        fire(0, 0)

        def it(i, carry):
            for k in range(2):
                g = 2 * i + k
                slot = k
                nxt = 1 - k

                @pl.when(g >= 1)
                def _():
                    wait_store(nxt)

                @pl.when(g + 1 < _HN_IT)
                def _():
                    fire(g + 1, nxt)

                drain_gathers(slot)
                pltpu.async_copy(rows[slot],
                                 out_hbm.at[pl.ds(row_base + g * CHUNK,
                                                  CHUNK)],
                                 sem_o[slot])
            return carry

        lax.fori_loop(0, _HN_IT // 2, it, 0)
        wait_store((_HN_IT - 1) % 2)

    return body


_gather_a = _build(0)
_gather_b = _build(1)


def kernel(input_seq, weights):
    idx = input_seq.reshape(TOTAL // G, G)
    w128 = _conv_in(weights.T)
    w_lin = w128.reshape(_NEMB_PAD, D)
    ga = _gather_a(idx, w_lin)
    gb = _gather_b(idx, w_lin)
    oa = _conv_out_a(ga.reshape(_HTOT * D // 128, 128))
    out5 = _conv_out_b(gb.reshape(_HTOT * D // 128, 128), oa)
    return out5.transpose(2, 4, 0, 1, 3).reshape(BATCH, HIST, D)


# speedup vs baseline: 1.3913x; 1.0038x over previous
import functools

import jax
import jax.numpy as jnp
from jax import lax
from jax.experimental import pallas as pl
from jax.experimental.pallas import tpu as pltpu
from jax.experimental.pallas import tpu_sc as plsc

NUM_EMB = 1000000
D = 32
BATCH = 4096
HIST = 200
TOTAL = BATCH * HIST

NC = 2
NS = 16
NW = NC * NS
PER_W = TOTAL // NW

G = 128
CHUNK = 1280
N_G = CHUNK // G
N_IT = PER_W // CHUNK
IDX_ROWS_W = PER_W // G

_CIN = 32768
_CIN_GRID = -(-NUM_EMB // _CIN)


_Q = _CIN // 4
_QSH = _Q.bit_length() - 1
_NEMB_PAD = _CIN_GRID * _CIN


def _conv_in_body(in_ref, o_ref):
    t = in_ref[...].T
    for k0 in range(4):
        o_ref[:, k0 * 32:(k0 + 1) * 32] = t[k0 * _Q:(k0 + 1) * _Q, :]


_conv_in = pl.pallas_call(
    _conv_in_body,
    grid=(_CIN_GRID,),
    in_specs=[pl.BlockSpec((D, _CIN), lambda i: (0, i))],
    out_specs=pl.BlockSpec((_Q, 128), lambda i: (i, 0)),
    out_shape=jax.ShapeDtypeStruct((_NEMB_PAD * D // 128, 128), jnp.float32),
)

_BROWS = TOTAL * D // 128 // NW


def _conv_out_body(in_ref, o_ref):
    def one(h4, carry):
        blk = in_ref[pl.Slice(h4, 128, HIST * D // 128), :]
        blkT = blk.T
        for j in range(4):
            o_ref[h4 * 4 + j, :, 0] = (
                blkT[j * 32:(j + 1) * 32, :].reshape(4, 8, 128))
        return carry

    lax.fori_loop(0, HIST // 4, one, 0)


_conv_out_a = pl.pallas_call(
    _conv_out_body,
    grid=(NW // 2,),
    in_specs=[pl.BlockSpec((_BROWS, 128), lambda i: (i, 0))],
    out_specs=pl.BlockSpec((HIST, 4, 1, 8, 128), lambda i: (0, 0, i, 0, 0)),
    out_shape=jax.ShapeDtypeStruct((HIST, 4, NW, 8, 128), jnp.float32),
)


def _conv_out_body_b(in_ref, prev_ref, o_ref):
    del prev_ref
    _conv_out_body(in_ref, o_ref)


_conv_out_b = pl.pallas_call(
    _conv_out_body_b,
    grid=(NW // 2,),
    in_specs=[pl.BlockSpec((_BROWS, 128), lambda i: (i, 0)),
              pl.BlockSpec(memory_space=pl.ANY)],
    out_specs=pl.BlockSpec((HIST, 4, 1, 8, 128),
                           lambda i: (0, 0, NW // 2 + i, 0, 0)),
    out_shape=jax.ShapeDtypeStruct((HIST, 4, NW, 8, 128), jnp.float32),
    input_output_aliases={1: 0},
)


_HTOT = TOTAL // 2
_HPER_W = _HTOT // NW
_HN_IT = _HPER_W // CHUNK
_HIDX_ROWS = _HPER_W // G


def _build(half):
    mesh = plsc.VectorSubcoreMesh(core_axis_name="c", subcore_axis_name="s")

    @functools.partial(
        pl.kernel,
        mesh=mesh,
        out_type=jax.ShapeDtypeStruct((_HTOT, D), jnp.float32),
        scratch_types=[
            pltpu.VMEM((_HIDX_ROWS, G), jnp.int32),
            pltpu.VMEM((CHUNK, D), jnp.float32),
            pltpu.VMEM((CHUNK, D), jnp.float32),
            pltpu.SemaphoreType.DMA,
            pltpu.SemaphoreType.DMA,
            pltpu.SemaphoreType.DMA,
            pltpu.SemaphoreType.DMA,
        ],
        compiler_params=pltpu.CompilerParams(use_tc_tiling_on_sc=False),
    )
    def body(idx_hbm, table_hbm, out_hbm, idx_all, rows0, rows1,
             sg0, sg1, so0, so1):
        rows = (rows0, rows1)
        sem_g = (sg0, sg1)
        sem_o = (so0, so1)
        wid = lax.axis_index("s") * NC + lax.axis_index("c")
        row_base = wid * _HPER_W
        idx_row0 = half * (_HTOT // G) + wid * _HIDX_ROWS

        pltpu.sync_copy(idx_hbm.at[pl.ds(idx_row0, _HIDX_ROWS)], idx_all)

        def permute_idx(r, carry):
            for j in range(G // 16):
                v = idx_all[r, pl.ds(j * 16, 16)]
                idx_all[r, pl.ds(j * 16, 16)] = (
                    (v & -_CIN) | ((v & (_Q - 1)) << 2) | ((v >> _QSH) & 3))
            return carry

        lax.fori_loop(0, _HIDX_ROWS, permute_idx, 0)

        def fire(g, slot):
            for j in range(N_G):
                pltpu.async_copy(table_hbm.at[idx_all.at[g * N_G + j]],
                                 rows[slot].at[pl.ds(j * G, G)],
                                 sem_g[slot])

        def drain_gathers(slot):
            for j in range(N_G):
                pltpu.make_async_copy(table_hbm.at[idx_all.at[0]],
                                      rows[slot].at[pl.ds(j * G, G)],
                                      sem_g[slot]).wait()

        def wait_store(slot):
            pltpu.make_async_copy(rows[slot],
                                  out_hbm.at[pl.ds(row_base, CHUNK)],
                                  sem_o[slot]).wait()

        fire(0, 0)

        def it(i, carry):
            for k in range(2):
                g = 2 * i + k
                slot = k
                nxt = 1 - k

                @pl.when(g >= 1)
                def _():
                    wait_store(nxt)

                @pl.when(g + 1 < _HN_IT)
                def _():
                    fire(g + 1, nxt)

                drain_gathers(slot)
                pltpu.async_copy(rows[slot],
                                 out_hbm.at[pl.ds(row_base + g * CHUNK,
                                                  CHUNK)],
                                 sem_o[slot])
            return carry

        lax.fori_loop(0, _HN_IT // 2, it, 0)
        wait_store((_HN_IT - 1) % 2)

    return body


_gather_a = _build(0)
_gather_b = _build(1)


def kernel(input_seq, weights):
    idx = input_seq.reshape(TOTAL // G, G)
    w128 = _conv_in(weights.T)
    w_lin = w128.reshape(_NEMB_PAD, D)
    ga = _gather_a(idx, w_lin)
    gb = _gather_b(idx, w_lin)
    oa = _conv_out_a(ga.reshape(_HTOT * D // 128, 128))
    out5 = _conv_out_b(gb.reshape(_HTOT * D // 128, 128), oa)
    return out5.transpose(2, 4, 0, 1, 3).reshape(BATCH, HIST, D)
